# baseline (speedup 1.0000x reference)
import functools

import jax
import jax.numpy as jnp
from jax import lax
from jax.experimental import pallas as pl
from jax.experimental.pallas import tpu as pltpu
from jax.experimental.pallas import tpu_sc as plsc

NUM_EMB = 1000000
EMB_DIM = 64
BASE_DIM = 128
BATCH = 16384
FIELDS = 26

_NROWS = BATCH * FIELDS
_IDXW = 128
_IDX_ROWS = _NROWS // _IDXW

_NC, _NS = 2, 16
_NW = _NC * _NS

_ROWS_PER_W = _NROWS // _NW
_CHUNK = 1024
_IDXR_PER_CHUNK = _CHUNK // _IDXW
_STEPS = _ROWS_PER_W // _CHUNK


def _sc_gather(table, idx2d):
  mesh = plsc.VectorSubcoreMesh(core_axis_name="c", subcore_axis_name="s")

  @functools.partial(
      pl.kernel,
      mesh=mesh,
      compiler_params=pltpu.CompilerParams(use_tc_tiling_on_sc=False),
      out_type=jax.ShapeDtypeStruct((_NROWS, EMB_DIM), jnp.float32),
      scratch_types=[
          pltpu.VMEM((_IDXR_PER_CHUNK, _IDXW), jnp.int32),
          pltpu.VMEM((_CHUNK, EMB_DIM), jnp.float32),
          pltpu.SemaphoreType.DMA,
      ],
  )
  def k(table_hbm, idx_hbm, out_hbm, idx_v, rows_v, sem):
    wid = lax.axis_index("s") * _NC + lax.axis_index("c")
    idx_row0 = wid * (_ROWS_PER_W // _IDXW)
    row0 = wid * _ROWS_PER_W

    def step(t, _):
      pltpu.sync_copy(
          idx_hbm.at[pl.ds(idx_row0 + t * _IDXR_PER_CHUNK, _IDXR_PER_CHUNK)],
          idx_v)
      copies = []
      for j in range(_IDXR_PER_CHUNK):
        copies.append(
            pltpu.async_copy(
                table_hbm.at[idx_v.at[j]],
                rows_v.at[pl.ds(j * _IDXW, _IDXW)],
                sem))
      for c in copies:
        c.wait()
      pltpu.sync_copy(rows_v, out_hbm.at[pl.ds(row0 + t * _CHUNK, _CHUNK)])
      return 0

    lax.fori_loop(0, _STEPS, step, 0)

  return k(table, idx2d)


_MM_BLK = 2048


def _mm_body(x_ref, w_ref, o_ref):
  o_ref[...] = lax.dot_general(
      x_ref[...], w_ref[...],
      dimension_numbers=(((1,), (1,)), ((), ())),
      preferred_element_type=jnp.float32)


def _tc_project(rows, W):
  return pl.pallas_call(
      _mm_body,
      grid=(_NROWS // _MM_BLK,),
      in_specs=[
          pl.BlockSpec((_MM_BLK, EMB_DIM), lambda i: (i, 0)),
          pl.BlockSpec((BASE_DIM, EMB_DIM), lambda i: (0, 0)),
      ],
      out_specs=pl.BlockSpec((_MM_BLK, BASE_DIM), lambda i: (i, 0)),
      out_shape=jax.ShapeDtypeStruct((_NROWS, BASE_DIM), jnp.float32),
  )(rows, W)


def kernel(input, table, W):
  idx2d = input.astype(jnp.int32).reshape(_IDX_ROWS, _IDXW)
  rows = _sc_gather(table, idx2d)
  out = _tc_project(rows, W)
  return out.reshape(BATCH, FIELDS, BASE_DIM)
